# Initial kernel scaffold; baseline (speedup 1.0000x reference)
import functools

import jax
import jax.numpy as jnp
from jax import lax
from jax.experimental import pallas as pl
from jax.experimental.pallas import tpu as pltpu
from jax.experimental.pallas import tpu_sc as plsc

N = 10000
D = 128
E = 320000
NTILES = 32
CHUNK = 128
NCHUNK = 79
EPAD = NTILES * NCHUNK * CHUNK
NP = N + 16
ROWS_PER_TILE = NP // 16


_BR = 500


def _proj_body(x_ref, w_ref, asrc_ref, adst_ref, h_ref, as_ref, ad_ref):
    h = jnp.dot(x_ref[...], w_ref[...], preferred_element_type=jnp.float32)
    h_ref[...] = h
    as_ref[...] = jnp.dot(h, asrc_ref[...], preferred_element_type=jnp.float32)
    ad_ref[...] = jnp.dot(h, adst_ref[...], preferred_element_type=jnp.float32)


def _proj(x, w, asrc, adst):
    grid = (N // _BR,)
    return pl.pallas_call(
        _proj_body,
        grid=grid,
        in_specs=[
            pl.BlockSpec((_BR, D), lambda i: (i, 0)),
            pl.BlockSpec((D, D), lambda i: (0, 0)),
            pl.BlockSpec((D, 1), lambda i: (0, 0)),
            pl.BlockSpec((D, 1), lambda i: (0, 0)),
        ],
        out_specs=[
            pl.BlockSpec((_BR, D), lambda i: (i, 0)),
            pl.BlockSpec((_BR, 1), lambda i: (i, 0)),
            pl.BlockSpec((_BR, 1), lambda i: (i, 0)),
        ],
        out_shape=[
            jax.ShapeDtypeStruct((N, D), jnp.float32),
            jax.ShapeDtypeStruct((N, 1), jnp.float32),
            jax.ShapeDtypeStruct((N, 1), jnp.float32),
        ],
    )(x, w, asrc, adst)


def _norm(o_ref, sp_ref, b_ref):
    s = jnp.sum(sp_ref[...], axis=0)
    acc = o_ref[0] + o_ref[1]
    z = acc * (1.0 / (s + 1e-16))[:, None] + b_ref[...]
    return jnp.maximum(z, 0.0)


def _mid_body(o_ref, sp_ref, b_ref, w_ref, asrc_ref, adst_ref,
              h_ref, as_ref, ad_ref):
    z = _norm(o_ref, sp_ref, b_ref)
    h = jnp.dot(z, w_ref[...], preferred_element_type=jnp.float32)
    h_ref[...] = h
    as_ref[...] = jnp.dot(h, asrc_ref[...], preferred_element_type=jnp.float32)
    ad_ref[...] = jnp.dot(h, adst_ref[...], preferred_element_type=jnp.float32)


def _mid(o, sp, b, w, asrc, adst):
    grid = (N // _BR,)
    return pl.pallas_call(
        _mid_body,
        grid=grid,
        in_specs=[
            pl.BlockSpec((2, _BR, D), lambda i: (0, i, 0)),
            pl.BlockSpec((NTILES, _BR), lambda i: (0, i)),
            pl.BlockSpec((1, D), lambda i: (0, 0)),
            pl.BlockSpec((D, D), lambda i: (0, 0)),
            pl.BlockSpec((D, 1), lambda i: (0, 0)),
            pl.BlockSpec((D, 1), lambda i: (0, 0)),
        ],
        out_specs=[
            pl.BlockSpec((_BR, D), lambda i: (i, 0)),
            pl.BlockSpec((_BR, 1), lambda i: (i, 0)),
            pl.BlockSpec((_BR, 1), lambda i: (i, 0)),
        ],
        out_shape=[
            jax.ShapeDtypeStruct((N, D), jnp.float32),
            jax.ShapeDtypeStruct((N, 1), jnp.float32),
            jax.ShapeDtypeStruct((N, 1), jnp.float32),
        ],
    )(o, sp, b, w, asrc, adst)


def _final_body(o_ref, sp_ref, b_ref, out_ref):
    out_ref[...] = _norm(o_ref, sp_ref, b_ref)


def _final(o, sp, b):
    grid = (N // _BR,)
    return pl.pallas_call(
        _final_body,
        grid=grid,
        in_specs=[
            pl.BlockSpec((2, _BR, D), lambda i: (0, i, 0)),
            pl.BlockSpec((NTILES, _BR), lambda i: (0, i)),
            pl.BlockSpec((1, D), lambda i: (0, 0)),
        ],
        out_specs=pl.BlockSpec((_BR, D), lambda i: (i, 0)),
        out_shape=jax.ShapeDtypeStruct((N, D), jnp.float32),
    )(o, sp, b)


_MESH = plsc.VectorSubcoreMesh(core_axis_name="c", subcore_axis_name="s")


def _edge_body(h_hbm, as_hbm, ad_hbm, src_hbm, dst_hbm,
               o_hbm, sp_hbm,
               as_v, ad_v, src_v, dst_v, ex_v, s_v, rows_v, shared, sem):
    c = lax.axis_index("c")
    t = lax.axis_index("s")
    w = c * 16 + t

    pltpu.sync_copy(as_hbm, as_v)
    pltpu.sync_copy(ad_hbm, ad_v)
    pltpu.sync_copy(src_hbm.at[w], src_v)
    pltpu.sync_copy(dst_hbm.at[w], dst_v)

    zero16 = jnp.zeros((16,), jnp.float32)

    def _zero_s(i, carry):
        s_v[pl.ds(i * 16, 16)] = zero16
        return carry

    lax.fori_loop(0, NP // 16, _zero_s, 0)

    def _zero_rows(i, carry):
        for cg in range(8):
            rows_v[i, pl.ds(cg * 16, 16)] = zero16
        return carry

    lax.fori_loop(0, CHUNK, _zero_rows, 0)

    base = t * ROWS_PER_TILE
    off = 0
    for sz in (128, 128, 128, 128, ROWS_PER_TILE - 512):
        pltpu.sync_copy(rows_v.at[pl.ds(0, sz)], shared.at[pl.ds(base + off, sz)])
        off += sz
    plsc.subcore_barrier()

    def _scalar_chunk(j, carry):
        def _sub(k, carry2):
            s16 = src_v[j, pl.ds(k * 16, 16)]
            d16 = dst_v[j, pl.ds(k * 16, 16)]
            av = plsc.load_gather(as_v, [s16])
            dv = plsc.load_gather(ad_v, [d16])
            e = av + dv
            e = jnp.where(e >= 0.0, e, 0.2 * e)
            exv = jnp.exp(e)
            ex_v[j, pl.ds(k * 16, 16)] = exv
            plsc.addupdate_scatter(s_v, [d16], exv)
            return carry2

        return lax.fori_loop(0, CHUNK // 16, _sub, carry)

    lax.fori_loop(0, NCHUNK, _scalar_chunk, 0)

    def _row_chunk(j, carry):
        pltpu.async_copy(h_hbm.at[src_v.at[j]], rows_v, sem).wait()

        def _scale_row(r, carry2):
            exb = plsc.load_gather(
                ex_v, [jnp.full((16,), j, jnp.int32), jnp.full((16,), r, jnp.int32)])
            for cg in range(8):
                sl = pl.ds(cg * 16, 16)
                rows_v[r, sl] = rows_v[r, sl] * exb
            return carry2

        lax.fori_loop(0, CHUNK, _scale_row, 0)
        pltpu.sync_copy(rows_v, shared.at[dst_v.at[j]], add=True)
        return carry

    lax.fori_loop(0, NCHUNK, _row_chunk, 0)
    plsc.subcore_barrier()

    pltpu.sync_copy(s_v, sp_hbm.at[w])
    off = 0
    for sz in (128, 128, 128, 128, ROWS_PER_TILE - 512):
        pltpu.sync_copy(shared.at[pl.ds(base + off, sz)],
                        o_hbm.at[c, pl.ds(base + off, sz)])
        off += sz


_edge_kernel = pl.kernel(
    _edge_body,
    out_type=[
        jax.ShapeDtypeStruct((2, NP, D), jnp.float32),
        jax.ShapeDtypeStruct((NTILES, NP), jnp.float32),
    ],
    mesh=_MESH,
    scratch_types=[
        pltpu.VMEM((N,), jnp.float32),
        pltpu.VMEM((N,), jnp.float32),
        pltpu.VMEM((NCHUNK, CHUNK), jnp.int32),
        pltpu.VMEM((NCHUNK, CHUNK), jnp.int32),
        pltpu.VMEM((NCHUNK, CHUNK), jnp.float32),
        pltpu.VMEM((NP,), jnp.float32),
        pltpu.VMEM((CHUNK, D), jnp.float32),
        pltpu.VMEM_SHARED((NP, D), jnp.float32),
        pltpu.SemaphoreType.DMA,
    ],
)


def kernel(x, edge_index, W1, a_src1, a_dst1, b1, W2, a_src2, a_dst2, b2):
    src = edge_index[0].astype(jnp.int32)
    dst = edge_index[1].astype(jnp.int32)
    pad = EPAD - E
    srcp = jnp.concatenate([src, jnp.zeros((pad,), jnp.int32)]).reshape(
        NTILES, NCHUNK, CHUNK)
    dstp = jnp.concatenate([dst, jnp.full((pad,), N, jnp.int32)]).reshape(
        NTILES, NCHUNK, CHUNK)

    h1, as1, ad1 = _proj(x, W1, a_src1.reshape(D, 1), a_dst1.reshape(D, 1))
    o1, sp1 = _edge_kernel(h1, as1.reshape(N), ad1.reshape(N), srcp, dstp)
    h2, as2, ad2 = _mid(o1, sp1, b1.reshape(1, D), W2,
                        a_src2.reshape(D, 1), a_dst2.reshape(D, 1))
    o2, sp2 = _edge_kernel(h2, as2.reshape(N), ad2.reshape(N), srcp, dstp)
    return _final(o2, sp2, b2.reshape(1, D))


# speedup vs baseline: 34.5377x; 34.5377x over previous
import functools

import jax
import jax.numpy as jnp
from jax import lax
from jax.experimental import pallas as pl
from jax.experimental.pallas import tpu as pltpu
from jax.experimental.pallas import tpu_sc as plsc

N = 10000
D = 128
E = 320000
NTILES = 32
CHUNK = 64
NREAL = E // CHUNK
NCHUNK = 158
TOTALC = NTILES * NCHUNK
NP = 10112
ROWS_PER_TILE = NP // 16
M0 = 210
M1 = 2 * NCHUNK - M0


def _proj_body(x_ref, w_ref, asrc_ref, adst_ref, h_ref, as_ref, ad_ref):
    h = jnp.dot(x_ref[...], w_ref[...], preferred_element_type=jnp.float32)
    h_ref[pl.ds(0, N), :] = h
    h_ref[pl.ds(N, NP - N), :] = jnp.zeros((NP - N, D), jnp.float32)
    as_ref[pl.ds(0, N), :] = jnp.dot(h, asrc_ref[...],
                                     preferred_element_type=jnp.float32)
    as_ref[pl.ds(N, NP - N), :] = jnp.zeros((NP - N, 1), jnp.float32)
    ad_ref[pl.ds(0, N), :] = jnp.dot(h, adst_ref[...],
                                     preferred_element_type=jnp.float32)
    ad_ref[pl.ds(N, NP - N), :] = jnp.zeros((NP - N, 1), jnp.float32)


def _proj(x, w, asrc, adst):
    return pl.pallas_call(
        _proj_body,
        out_shape=[
            jax.ShapeDtypeStruct((NP, D), jnp.float32),
            jax.ShapeDtypeStruct((NP, 1), jnp.float32),
            jax.ShapeDtypeStruct((NP, 1), jnp.float32),
        ],
    )(x, w, asrc, adst)


def _norm(o_ref, sp_ref, b_ref):
    s = jnp.sum(sp_ref[...], axis=0)
    acc = o_ref[0] + o_ref[1]
    z = acc * (1.0 / (s + 1e-16))[:, None] + b_ref[...]
    return jnp.maximum(z, 0.0)


def _mid_body(o_ref, sp_ref, b_ref, w_ref, asrc_ref, adst_ref,
              h_ref, as_ref, ad_ref):
    z = _norm(o_ref, sp_ref, b_ref)
    h = jnp.dot(z, w_ref[...], preferred_element_type=jnp.float32)
    h_ref[...] = h
    as_ref[...] = jnp.dot(h, asrc_ref[...], preferred_element_type=jnp.float32)
    ad_ref[...] = jnp.dot(h, adst_ref[...], preferred_element_type=jnp.float32)


def _mid(o, sp, b, w, asrc, adst):
    return pl.pallas_call(
        _mid_body,
        out_shape=[
            jax.ShapeDtypeStruct((NP, D), jnp.float32),
            jax.ShapeDtypeStruct((NP, 1), jnp.float32),
            jax.ShapeDtypeStruct((NP, 1), jnp.float32),
        ],
    )(o, sp, b, w, asrc, adst)


def _final_body(o_ref, sp_ref, b_ref, out_ref):
    out_ref[...] = _norm(o_ref, sp_ref, b_ref)[0:N, :]


def _final(o, sp, b):
    return pl.pallas_call(
        _final_body,
        out_shape=jax.ShapeDtypeStruct((N, D), jnp.float32),
    )(o, sp, b)


_MESH = plsc.VectorSubcoreMesh(core_axis_name="c", subcore_axis_name="s")


def _edge_body(h_hbm, as_hbm, ad_hbm, ei_hbm,
               o_hbm, sp_hbm,
               as_v, ad_v, src_a, src_b, dst_a, dst_b, ex_a, ex_b,
               s_v, rows_a, rows_b, shared,
               sem_sa, sem_sb, sem_da, sem_db, sem_ga, sem_gb,
               sem_ca, sem_cb):
    c = lax.axis_index("c")
    t = lax.axis_index("s")
    w = c * 16 + t
    start = jnp.where(c == 0, t * M0, 16 * M0 + t * M1)
    npairs = jnp.where(c == 0, M0 // 2, M1 // 2)
    m = jnp.where(c == 0, M0, M1)

    pltpu.sync_copy(as_hbm, as_v)
    pltpu.sync_copy(ad_hbm, ad_v)

    zero16 = jnp.zeros((16,), jnp.float32)
    zero16i = jnp.zeros((16,), jnp.int32)
    fullN = jnp.full((16,), N, jnp.int32)

    def _zero_s(i, carry):
        s_v[pl.ds(i * 16, 16)] = zero16
        return carry

    lax.fori_loop(0, NP // 16, _zero_s, 0)

    def _zero_rows(i, carry):
        for cg in range(8):
            rows_a[i, pl.ds(cg * 16, 16)] = zero16
            rows_b[i, pl.ds(cg * 16, 16)] = zero16
        return carry

    lax.fori_loop(0, CHUNK, _zero_rows, 0)

    base = t * ROWS_PER_TILE
    off = 0
    for sz in (64,) * 9 + (ROWS_PER_TILE - 576,):
        pltpu.sync_copy(rows_a.at[pl.ds(0, sz)], shared.at[pl.ds(base + off, sz)])
        off += sz
    plsc.subcore_barrier()

    def _launch_idx(q, row, buf, sem):
        @pl.when(q < NREAL)
        def _():
            pltpu.async_copy(ei_hbm.at[row, pl.ds(q * CHUNK, CHUNK)], buf, sem)

    def _wait_idx(q, row, buf, sem, fill):
        @pl.when(q < NREAL)
        def _():
            pltpu.make_async_copy(
                ei_hbm.at[row, pl.ds(q * CHUNK, CHUNK)], buf, sem).wait()

        @pl.when(q >= NREAL)
        def _():
            for k in range(CHUNK // 16):
                buf[pl.ds(k * 16, 16)] = fill

    def _ex_compute(src_c, dst_c, ex_c):
        for k in range(CHUNK // 16):
            sl = pl.ds(k * 16, 16)
            s16 = src_c[sl]
            d16 = dst_c[sl]
            e = plsc.load_gather(as_v, [s16]) + plsc.load_gather(ad_v, [d16])
            e = jnp.where(e >= 0.0, e, 0.2 * e)
            exv = jnp.exp(e)
            ex_c[sl] = exv
            plsc.addupdate_scatter(s_v, [d16], exv)

    def _multiply(rows, ex_c):
        @plsc.parallel_loop(0, CHUNK, 1, unroll=8)
        def _rowfn(rr):
            exb = plsc.load_gather(ex_c, [jnp.full((16,), rr, jnp.int32)])
            for cg in range(8):
                sl = pl.ds(cg * 16, 16)
                rows[rr, sl] = rows[rr, sl] * exb

    for k in range(CHUNK // 16):
        dst_b[pl.ds(k * 16, 16)] = fullN
    pltpu.async_copy(rows_b, shared.at[dst_b], sem_cb, add=True)
    pltpu.sync_copy(ei_hbm.at[0, pl.ds(start * CHUNK, CHUNK)], src_a)
    pltpu.sync_copy(ei_hbm.at[1, pl.ds(start * CHUNK, CHUNK)], dst_a)
    pltpu.async_copy(h_hbm.at[src_a], rows_a, sem_ga)

    def _pair(i, carry):
        j0 = start + 2 * i
        _launch_idx(j0 + 1, 0, src_b, sem_sb)
        _ex_compute(src_a, dst_a, ex_a)
        pltpu.make_async_copy(rows_b, shared.at[dst_b], sem_cb).wait()
        _launch_idx(j0 + 1, 1, dst_b, sem_db)
        _wait_idx(j0 + 1, 0, src_b, sem_sb, zero16i)
        pltpu.async_copy(h_hbm.at[src_b], rows_b, sem_gb)
        pltpu.make_async_copy(h_hbm.at[src_a], rows_a, sem_ga).wait()
        _multiply(rows_a, ex_a)
        pltpu.async_copy(rows_a, shared.at[dst_a], sem_ca, add=True)

        @pl.when(j0 + 2 < start + m)
        def _():
            _launch_idx(j0 + 2, 0, src_a, sem_sa)

        _wait_idx(j0 + 1, 1, dst_b, sem_db, fullN)
        _ex_compute(src_b, dst_b, ex_b)
        pltpu.make_async_copy(rows_a, shared.at[dst_a], sem_ca).wait()

        @pl.when(j0 + 2 < start + m)
        def _():
            _launch_idx(j0 + 2, 1, dst_a, sem_da)
            _wait_idx(j0 + 2, 0, src_a, sem_sa, zero16i)
            pltpu.async_copy(h_hbm.at[src_a], rows_a, sem_ga)

        pltpu.make_async_copy(h_hbm.at[src_b], rows_b, sem_gb).wait()
        _multiply(rows_b, ex_b)
        pltpu.async_copy(rows_b, shared.at[dst_b], sem_cb, add=True)

        @pl.when(j0 + 2 < start + m)
        def _():
            _wait_idx(j0 + 2, 1, dst_a, sem_da, fullN)

        return carry

    lax.fori_loop(0, npairs, _pair, 0)
    pltpu.make_async_copy(rows_b, shared.at[dst_b], sem_cb).wait()
    plsc.subcore_barrier()

    pltpu.sync_copy(s_v, sp_hbm.at[w])
    off = 0
    for sz in (128, 128, 128, 128, ROWS_PER_TILE - 512):
        pltpu.sync_copy(shared.at[pl.ds(base + off, sz)],
                        o_hbm.at[c, pl.ds(base + off, sz)])
        off += sz


_edge_kernel = pl.kernel(
    _edge_body,
    out_type=[
        jax.ShapeDtypeStruct((2, NP, D), jnp.float32),
        jax.ShapeDtypeStruct((NTILES, NP), jnp.float32),
    ],
    mesh=_MESH,
    compiler_params=pltpu.CompilerParams(needs_layout_passes=False),
    scratch_types=[
        pltpu.VMEM((NP,), jnp.float32),
        pltpu.VMEM((NP,), jnp.float32),
        pltpu.VMEM((CHUNK,), jnp.int32),
        pltpu.VMEM((CHUNK,), jnp.int32),
        pltpu.VMEM((CHUNK,), jnp.int32),
        pltpu.VMEM((CHUNK,), jnp.int32),
        pltpu.VMEM((CHUNK,), jnp.float32),
        pltpu.VMEM((CHUNK,), jnp.float32),
        pltpu.VMEM((NP,), jnp.float32),
        pltpu.VMEM((CHUNK, D), jnp.float32),
        pltpu.VMEM((CHUNK, D), jnp.float32),
        pltpu.VMEM_SHARED((NP, D), jnp.float32),
        pltpu.SemaphoreType.DMA,
        pltpu.SemaphoreType.DMA,
        pltpu.SemaphoreType.DMA,
        pltpu.SemaphoreType.DMA,
        pltpu.SemaphoreType.DMA,
        pltpu.SemaphoreType.DMA,
        pltpu.SemaphoreType.DMA,
        pltpu.SemaphoreType.DMA,
    ],
)


def kernel(x, edge_index, W1, a_src1, a_dst1, b1, W2, a_src2, a_dst2, b2):
    ei = edge_index.astype(jnp.int32)
    h1, as1, ad1 = _proj(x, W1, a_src1.reshape(D, 1), a_dst1.reshape(D, 1))
    o1, sp1 = _edge_kernel(h1, as1.reshape(NP), ad1.reshape(NP), ei)
    h2, as2, ad2 = _mid(o1, sp1, b1.reshape(1, D), W2,
                        a_src2.reshape(D, 1), a_dst2.reshape(D, 1))
    o2, sp2 = _edge_kernel(h2, as2.reshape(NP), ad2.reshape(NP), ei)
    return _final(o2, sp2, b2.reshape(1, D))


# speedup vs baseline: 36.6170x; 1.0602x over previous
import functools

import jax
import jax.numpy as jnp
from jax import lax
from jax.experimental import pallas as pl
from jax.experimental.pallas import tpu as pltpu
from jax.experimental.pallas import tpu_sc as plsc

N = 10000
D = 128
E = 320000
NTILES = 32
CHUNK = 64
NREAL = E // CHUNK
NCHUNK = 158
TOTALC = NTILES * NCHUNK
NP = 10112
ROWS_PER_TILE = NP // 16
M0 = 218
M1 = 2 * NCHUNK - M0


def _proj_body(x_ref, w_ref, asrc_ref, adst_ref, h_ref, as_ref, ad_ref):
    h = jnp.dot(x_ref[...], w_ref[...], preferred_element_type=jnp.float32)
    h_ref[pl.ds(0, N), :] = h
    h_ref[pl.ds(N, NP - N), :] = jnp.zeros((NP - N, D), jnp.float32)
    as_ref[pl.ds(0, N)] = jnp.sum(h * asrc_ref[...], axis=1)
    as_ref[pl.ds(N, NP - N)] = jnp.zeros((NP - N,), jnp.float32)
    ad_ref[pl.ds(0, N)] = jnp.sum(h * adst_ref[...], axis=1)
    ad_ref[pl.ds(N, NP - N)] = jnp.zeros((NP - N,), jnp.float32)


def _proj(x, w, asrc, adst):
    return pl.pallas_call(
        _proj_body,
        out_shape=[
            jax.ShapeDtypeStruct((NP, D), jnp.float32),
            jax.ShapeDtypeStruct((NP,), jnp.float32),
            jax.ShapeDtypeStruct((NP,), jnp.float32),
        ],
    )(x, w, asrc, adst)


def _norm(o_ref, sp_ref, b_ref):
    s = jnp.sum(sp_ref[...], axis=0)
    acc = o_ref[0] + o_ref[1]
    z = acc * (1.0 / (s + 1e-16))[:, None] + b_ref[...]
    return jnp.maximum(z, 0.0)


def _mid_body(o_ref, sp_ref, b_ref, w_ref, asrc_ref, adst_ref,
              h_ref, as_ref, ad_ref):
    z = _norm(o_ref, sp_ref, b_ref)
    h = jnp.dot(z, w_ref[...], preferred_element_type=jnp.float32)
    h_ref[...] = h
    as_ref[...] = jnp.sum(h * asrc_ref[...], axis=1)
    ad_ref[...] = jnp.sum(h * adst_ref[...], axis=1)


def _mid(o, sp, b, w, asrc, adst):
    return pl.pallas_call(
        _mid_body,
        out_shape=[
            jax.ShapeDtypeStruct((NP, D), jnp.float32),
            jax.ShapeDtypeStruct((NP,), jnp.float32),
            jax.ShapeDtypeStruct((NP,), jnp.float32),
        ],
    )(o, sp, b, w, asrc, adst)


def _final_body(o_ref, sp_ref, b_ref, out_ref):
    out_ref[...] = _norm(o_ref, sp_ref, b_ref)[0:N, :]


def _final(o, sp, b):
    return pl.pallas_call(
        _final_body,
        out_shape=jax.ShapeDtypeStruct((N, D), jnp.float32),
    )(o, sp, b)


_MESH = plsc.VectorSubcoreMesh(core_axis_name="c", subcore_axis_name="s")


def _edge_body(h_hbm, as_hbm, ad_hbm, ei_hbm,
               o_hbm, sp_hbm,
               as_v, ad_v, src_a, src_b, dst_a, dst_b, ex_a, ex_b,
               s_v, rows_a, rows_b, shared,
               sem_sa, sem_sb, sem_da, sem_db, sem_ga, sem_gb,
               sem_ca, sem_cb):
    c = lax.axis_index("c")
    t = lax.axis_index("s")
    w = c * 16 + t
    start = jnp.where(c == 0, t * M0, 16 * M0 + t * M1)
    npairs = jnp.where(c == 0, M0 // 2, M1 // 2)
    m = jnp.where(c == 0, M0, M1)

    pltpu.sync_copy(as_hbm, as_v)
    pltpu.sync_copy(ad_hbm, ad_v)

    zero16 = jnp.zeros((16,), jnp.float32)
    zero16i = jnp.zeros((16,), jnp.int32)
    fullN = jnp.full((16,), N, jnp.int32)

    def _zero_s(i, carry):
        s_v[pl.ds(i * 16, 16)] = zero16
        return carry

    lax.fori_loop(0, NP // 16, _zero_s, 0)

    def _zero_rows(i, carry):
        for cg in range(8):
            rows_a[i, pl.ds(cg * 16, 16)] = zero16
            rows_b[i, pl.ds(cg * 16, 16)] = zero16
        return carry

    lax.fori_loop(0, CHUNK, _zero_rows, 0)

    base = t * ROWS_PER_TILE
    off = 0
    for sz in (64,) * 9 + (ROWS_PER_TILE - 576,):
        pltpu.sync_copy(rows_a.at[pl.ds(0, sz)], shared.at[pl.ds(base + off, sz)])
        off += sz
    plsc.subcore_barrier()

    def _launch_idx(q, row, buf, sem):
        @pl.when(q < NREAL)
        def _():
            pltpu.async_copy(ei_hbm.at[row, pl.ds(q * CHUNK, CHUNK)], buf, sem)

    def _wait_idx(q, row, buf, sem, fill):
        @pl.when(q < NREAL)
        def _():
            pltpu.make_async_copy(
                ei_hbm.at[row, pl.ds(q * CHUNK, CHUNK)], buf, sem).wait()

        @pl.when(q >= NREAL)
        def _():
            for k in range(CHUNK // 16):
                buf[pl.ds(k * 16, 16)] = fill

    def _ex_compute(src_c, dst_c, ex_c):
        for k in range(CHUNK // 16):
            sl = pl.ds(k * 16, 16)
            s16 = src_c[sl]
            d16 = dst_c[sl]
            e = plsc.load_gather(as_v, [s16]) + plsc.load_gather(ad_v, [d16])
            e = jnp.where(e >= 0.0, e, 0.2 * e)
            exv = jnp.exp(e)
            ex_c[sl] = exv
            plsc.addupdate_scatter(s_v, [d16], exv)

    def _multiply(rows, ex_c):
        @plsc.parallel_loop(0, CHUNK, 1, unroll=8)
        def _rowfn(rr):
            exb = plsc.load_gather(ex_c, [jnp.full((16,), rr, jnp.int32)])
            for cg in range(8):
                sl = pl.ds(cg * 16, 16)
                rows[rr, sl] = rows[rr, sl] * exb

    for k in range(CHUNK // 16):
        dst_b[pl.ds(k * 16, 16)] = fullN
    pltpu.async_copy(rows_b, shared.at[dst_b], sem_cb, add=True)
    pltpu.sync_copy(ei_hbm.at[0, pl.ds(start * CHUNK, CHUNK)], src_a)
    pltpu.sync_copy(ei_hbm.at[1, pl.ds(start * CHUNK, CHUNK)], dst_a)
    pltpu.async_copy(h_hbm.at[src_a], rows_a, sem_ga)

    def _pair(i, carry):
        j0 = start + 2 * i
        _launch_idx(j0 + 1, 0, src_b, sem_sb)
        _ex_compute(src_a, dst_a, ex_a)
        pltpu.make_async_copy(rows_b, shared.at[dst_b], sem_cb).wait()
        _launch_idx(j0 + 1, 1, dst_b, sem_db)
        _wait_idx(j0 + 1, 0, src_b, sem_sb, zero16i)
        pltpu.async_copy(h_hbm.at[src_b], rows_b, sem_gb)
        pltpu.make_async_copy(h_hbm.at[src_a], rows_a, sem_ga).wait()
        _multiply(rows_a, ex_a)
        pltpu.async_copy(rows_a, shared.at[dst_a], sem_ca, add=True)

        @pl.when(j0 + 2 < start + m)
        def _():
            _launch_idx(j0 + 2, 0, src_a, sem_sa)

        _wait_idx(j0 + 1, 1, dst_b, sem_db, fullN)
        _ex_compute(src_b, dst_b, ex_b)
        pltpu.make_async_copy(rows_a, shared.at[dst_a], sem_ca).wait()

        @pl.when(j0 + 2 < start + m)
        def _():
            _launch_idx(j0 + 2, 1, dst_a, sem_da)
            _wait_idx(j0 + 2, 0, src_a, sem_sa, zero16i)
            pltpu.async_copy(h_hbm.at[src_a], rows_a, sem_ga)

        pltpu.make_async_copy(h_hbm.at[src_b], rows_b, sem_gb).wait()
        _multiply(rows_b, ex_b)
        pltpu.async_copy(rows_b, shared.at[dst_b], sem_cb, add=True)

        @pl.when(j0 + 2 < start + m)
        def _():
            _wait_idx(j0 + 2, 1, dst_a, sem_da, fullN)

        return carry

    lax.fori_loop(0, npairs, _pair, 0)
    pltpu.make_async_copy(rows_b, shared.at[dst_b], sem_cb).wait()
    plsc.subcore_barrier()

    pltpu.sync_copy(s_v, sp_hbm.at[w])
    off = 0
    for sz in (128, 128, 128, 128, ROWS_PER_TILE - 512):
        pltpu.sync_copy(shared.at[pl.ds(base + off, sz)],
                        o_hbm.at[c, pl.ds(base + off, sz)])
        off += sz


_edge_kernel = pl.kernel(
    _edge_body,
    out_type=[
        jax.ShapeDtypeStruct((2, NP, D), jnp.float32),
        jax.ShapeDtypeStruct((NTILES, NP), jnp.float32),
    ],
    mesh=_MESH,
    compiler_params=pltpu.CompilerParams(needs_layout_passes=False),
    scratch_types=[
        pltpu.VMEM((NP,), jnp.float32),
        pltpu.VMEM((NP,), jnp.float32),
        pltpu.VMEM((CHUNK,), jnp.int32),
        pltpu.VMEM((CHUNK,), jnp.int32),
        pltpu.VMEM((CHUNK,), jnp.int32),
        pltpu.VMEM((CHUNK,), jnp.int32),
        pltpu.VMEM((CHUNK,), jnp.float32),
        pltpu.VMEM((CHUNK,), jnp.float32),
        pltpu.VMEM((NP,), jnp.float32),
        pltpu.VMEM((CHUNK, D), jnp.float32),
        pltpu.VMEM((CHUNK, D), jnp.float32),
        pltpu.VMEM_SHARED((NP, D), jnp.float32),
        pltpu.SemaphoreType.DMA,
        pltpu.SemaphoreType.DMA,
        pltpu.SemaphoreType.DMA,
        pltpu.SemaphoreType.DMA,
        pltpu.SemaphoreType.DMA,
        pltpu.SemaphoreType.DMA,
        pltpu.SemaphoreType.DMA,
        pltpu.SemaphoreType.DMA,
    ],
)


def kernel(x, edge_index, W1, a_src1, a_dst1, b1, W2, a_src2, a_dst2, b2):
    ei = edge_index.astype(jnp.int32)
    h1, as1, ad1 = _proj(x, W1, a_src1.reshape(1, D), a_dst1.reshape(1, D))
    o1, sp1 = _edge_kernel(h1, as1, ad1, ei)
    h2, as2, ad2 = _mid(o1, sp1, b1.reshape(1, D), W2,
                        a_src2.reshape(1, D), a_dst2.reshape(1, D))
    o2, sp2 = _edge_kernel(h2, as2, ad2, ei)
    return _final(o2, sp2, b2.reshape(1, D))


# speedup vs baseline: 37.2333x; 1.0168x over previous
import functools

import jax
import jax.numpy as jnp
from jax import lax
from jax.experimental import pallas as pl
from jax.experimental.pallas import tpu as pltpu
from jax.experimental.pallas import tpu_sc as plsc

N = 10000
D = 128
E = 320000
NTILES = 32
CHUNK = 64
NREAL = E // CHUNK
NCHUNK = 158
TOTALC = NTILES * NCHUNK
NP = 10112
ROWS_PER_TILE = NP // 16
M0 = 214
M1 = 2 * NCHUNK - M0


def _proj_body(x_ref, w_ref, asrc_ref, adst_ref, h_ref, as_ref, ad_ref):
    h = jnp.dot(x_ref[...], w_ref[...], preferred_element_type=jnp.float32)
    h_ref[pl.ds(0, N), :] = h
    h_ref[pl.ds(N, NP - N), :] = jnp.zeros((NP - N, D), jnp.float32)
    as_ref[pl.ds(0, N)] = jnp.sum(h * asrc_ref[...], axis=1)
    as_ref[pl.ds(N, NP - N)] = jnp.zeros((NP - N,), jnp.float32)
    ad_ref[pl.ds(0, N)] = jnp.sum(h * adst_ref[...], axis=1)
    ad_ref[pl.ds(N, NP - N)] = jnp.zeros((NP - N,), jnp.float32)


def _proj(x, w, asrc, adst):
    return pl.pallas_call(
        _proj_body,
        out_shape=[
            jax.ShapeDtypeStruct((NP, D), jnp.float32),
            jax.ShapeDtypeStruct((NP,), jnp.float32),
            jax.ShapeDtypeStruct((NP,), jnp.float32),
        ],
    )(x, w, asrc, adst)


def _norm(o_ref, sp_ref, b_ref):
    s = jnp.sum(sp_ref[...], axis=0)
    acc = o_ref[0] + o_ref[1]
    z = acc * (1.0 / (s + 1e-16))[:, None] + b_ref[...]
    return jnp.maximum(z, 0.0)


def _mid_body(o_ref, sp_ref, b_ref, w_ref, asrc_ref, adst_ref,
              h_ref, as_ref, ad_ref):
    z = _norm(o_ref, sp_ref, b_ref)
    h = jnp.dot(z, w_ref[...], preferred_element_type=jnp.float32)
    h_ref[...] = h
    as_ref[...] = jnp.sum(h * asrc_ref[...], axis=1)
    ad_ref[...] = jnp.sum(h * adst_ref[...], axis=1)


def _mid(o, sp, b, w, asrc, adst):
    return pl.pallas_call(
        _mid_body,
        out_shape=[
            jax.ShapeDtypeStruct((NP, D), jnp.float32),
            jax.ShapeDtypeStruct((NP,), jnp.float32),
            jax.ShapeDtypeStruct((NP,), jnp.float32),
        ],
    )(o, sp, b, w, asrc, adst)


def _final_body(o_ref, sp_ref, b_ref, out_ref):
    out_ref[...] = _norm(o_ref, sp_ref, b_ref)[0:N, :]


def _final(o, sp, b):
    return pl.pallas_call(
        _final_body,
        out_shape=jax.ShapeDtypeStruct((N, D), jnp.float32),
    )(o, sp, b)


_MESH = plsc.VectorSubcoreMesh(core_axis_name="c", subcore_axis_name="s")


def _edge_body(h_hbm, as_hbm, ad_hbm, ei_hbm,
               o_hbm, sp_hbm,
               as_v, ad_v, src_a, src_b, dst_a, dst_b, ex_a, ex_b,
               s_v, rows_a, rows_b, shared,
               sem_sa, sem_sb, sem_da, sem_db, sem_ga, sem_gb,
               sem_ca, sem_cb):
    c = lax.axis_index("c")
    t = lax.axis_index("s")
    w = c * 16 + t
    start = jnp.where(c == 0, t * M0, 16 * M0 + t * M1)
    npairs = jnp.where(c == 0, M0 // 2, M1 // 2)
    m = jnp.where(c == 0, M0, M1)

    pltpu.sync_copy(as_hbm, as_v)
    pltpu.sync_copy(ad_hbm, ad_v)

    zero16 = jnp.zeros((16,), jnp.float32)
    zero16i = jnp.zeros((16,), jnp.int32)
    fullN = jnp.full((16,), N, jnp.int32)

    def _zero_s(i, carry):
        s_v[pl.ds(i * 16, 16)] = zero16
        return carry

    lax.fori_loop(0, NP // 16, _zero_s, 0)

    def _zero_rows(i, carry):
        for cg in range(8):
            rows_a[i, pl.ds(cg * 16, 16)] = zero16
            rows_b[i, pl.ds(cg * 16, 16)] = zero16
        return carry

    lax.fori_loop(0, CHUNK, _zero_rows, 0)

    base = t * ROWS_PER_TILE
    off = 0
    for sz in (64,) * 9 + (ROWS_PER_TILE - 576,):
        pltpu.sync_copy(rows_a.at[pl.ds(0, sz)], shared.at[pl.ds(base + off, sz)])
        off += sz
    plsc.subcore_barrier()

    def _launch_idx(q, row, buf, sem):
        @pl.when(q < NREAL)
        def _():
            pltpu.async_copy(ei_hbm.at[row, pl.ds(q * CHUNK, CHUNK)], buf, sem)

    def _wait_idx(q, row, buf, sem, fill):
        @pl.when(q < NREAL)
        def _():
            pltpu.make_async_copy(
                ei_hbm.at[row, pl.ds(q * CHUNK, CHUNK)], buf, sem).wait()

        @pl.when(q >= NREAL)
        def _():
            for k in range(CHUNK // 16):
                buf[pl.ds(k * 16, 16)] = fill

    def _ex_compute(src_c, dst_c, ex_c):
        for k in range(CHUNK // 16):
            sl = pl.ds(k * 16, 16)
            s16 = src_c[sl]
            d16 = dst_c[sl]
            e = plsc.load_gather(as_v, [s16]) + plsc.load_gather(ad_v, [d16])
            e = jnp.where(e >= 0.0, e, 0.2 * e)
            exv = jnp.exp(e)
            ex_c[sl] = exv
            plsc.addupdate_scatter(s_v, [d16], exv)

    def _multiply(rows, ex_c):
        @plsc.parallel_loop(0, CHUNK, 1, unroll=8)
        def _rowfn(rr):
            exb = plsc.load_gather(ex_c, [jnp.full((16,), rr, jnp.int32)])
            for cg in range(8):
                sl = pl.ds(cg * 16, 16)
                rows[rr, sl] = rows[rr, sl] * exb

    for k in range(CHUNK // 16):
        dst_b[pl.ds(k * 16, 16)] = fullN
    pltpu.async_copy(rows_b, shared.at[dst_b], sem_cb, add=True)
    pltpu.sync_copy(ei_hbm.at[0, pl.ds(start * CHUNK, CHUNK)], src_a)
    pltpu.sync_copy(ei_hbm.at[1, pl.ds(start * CHUNK, CHUNK)], dst_a)
    pltpu.async_copy(h_hbm.at[src_a], rows_a, sem_ga)

    def _pair(i, carry):
        j0 = start + 2 * i
        _launch_idx(j0 + 1, 0, src_b, sem_sb)
        _ex_compute(src_a, dst_a, ex_a)
        pltpu.make_async_copy(rows_b, shared.at[dst_b], sem_cb).wait()
        _launch_idx(j0 + 1, 1, dst_b, sem_db)
        _wait_idx(j0 + 1, 0, src_b, sem_sb, zero16i)
        pltpu.async_copy(h_hbm.at[src_b], rows_b, sem_gb)
        pltpu.make_async_copy(h_hbm.at[src_a], rows_a, sem_ga).wait()
        _multiply(rows_a, ex_a)
        pltpu.async_copy(rows_a, shared.at[dst_a], sem_ca, add=True)

        @pl.when(j0 + 2 < start + m)
        def _():
            _launch_idx(j0 + 2, 0, src_a, sem_sa)

        _wait_idx(j0 + 1, 1, dst_b, sem_db, fullN)
        _ex_compute(src_b, dst_b, ex_b)
        pltpu.make_async_copy(rows_a, shared.at[dst_a], sem_ca).wait()

        @pl.when(j0 + 2 < start + m)
        def _():
            _launch_idx(j0 + 2, 1, dst_a, sem_da)
            _wait_idx(j0 + 2, 0, src_a, sem_sa, zero16i)
            pltpu.async_copy(h_hbm.at[src_a], rows_a, sem_ga)

        pltpu.make_async_copy(h_hbm.at[src_b], rows_b, sem_gb).wait()
        _multiply(rows_b, ex_b)
        pltpu.async_copy(rows_b, shared.at[dst_b], sem_cb, add=True)

        @pl.when(j0 + 2 < start + m)
        def _():
            _wait_idx(j0 + 2, 1, dst_a, sem_da, fullN)

        return carry

    lax.fori_loop(0, npairs, _pair, 0)
    pltpu.make_async_copy(rows_b, shared.at[dst_b], sem_cb).wait()
    plsc.subcore_barrier()

    pltpu.sync_copy(s_v, sp_hbm.at[w])
    off = 0
    for sz in (128, 128, 128, 128, ROWS_PER_TILE - 512):
        pltpu.sync_copy(shared.at[pl.ds(base + off, sz)],
                        o_hbm.at[c, pl.ds(base + off, sz)])
        off += sz


_edge_kernel = pl.kernel(
    _edge_body,
    out_type=[
        jax.ShapeDtypeStruct((2, NP, D), jnp.float32),
        jax.ShapeDtypeStruct((NTILES, NP), jnp.float32),
    ],
    mesh=_MESH,
    compiler_params=pltpu.CompilerParams(needs_layout_passes=False),
    scratch_types=[
        pltpu.VMEM((NP,), jnp.float32),
        pltpu.VMEM((NP,), jnp.float32),
        pltpu.VMEM((CHUNK,), jnp.int32),
        pltpu.VMEM((CHUNK,), jnp.int32),
        pltpu.VMEM((CHUNK,), jnp.int32),
        pltpu.VMEM((CHUNK,), jnp.int32),
        pltpu.VMEM((CHUNK,), jnp.float32),
        pltpu.VMEM((CHUNK,), jnp.float32),
        pltpu.VMEM((NP,), jnp.float32),
        pltpu.VMEM((CHUNK, D), jnp.float32),
        pltpu.VMEM((CHUNK, D), jnp.float32),
        pltpu.VMEM_SHARED((NP, D), jnp.float32),
        pltpu.SemaphoreType.DMA,
        pltpu.SemaphoreType.DMA,
        pltpu.SemaphoreType.DMA,
        pltpu.SemaphoreType.DMA,
        pltpu.SemaphoreType.DMA,
        pltpu.SemaphoreType.DMA,
        pltpu.SemaphoreType.DMA,
        pltpu.SemaphoreType.DMA,
    ],
)


def kernel(x, edge_index, W1, a_src1, a_dst1, b1, W2, a_src2, a_dst2, b2):
    ei = edge_index.astype(jnp.int32)
    h1, as1, ad1 = _proj(x, W1, a_src1.reshape(1, D), a_dst1.reshape(1, D))
    o1, sp1 = _edge_kernel(h1, as1, ad1, ei)
    h2, as2, ad2 = _mid(o1, sp1, b1.reshape(1, D), W2,
                        a_src2.reshape(1, D), a_dst2.reshape(1, D))
    o2, sp2 = _edge_kernel(h2, as2, ad2, ei)
    return _final(o2, sp2, b2.reshape(1, D))


# speedup vs baseline: 37.2351x; 1.0000x over previous
import jax
import jax.numpy as jnp
from jax import lax
from jax.experimental import pallas as pl
from jax.experimental.pallas import tpu as pltpu
from jax.experimental.pallas import tpu_sc as plsc

N = 10000
D = 128
E = 320000
NTILES = 32
CHUNK = 64
NREAL = E // CHUNK
NCHUNK = 158
TOTALC = NTILES * NCHUNK
NP = 10112
ROWS_PER_TILE = NP // 16
M0 = 214
M1 = 2 * NCHUNK - M0


def _proj_body(x_ref, w_ref, asrc_ref, adst_ref, h_ref, as_ref, ad_ref):
    h = jnp.dot(x_ref[...], w_ref[...], preferred_element_type=jnp.float32)
    h_ref[pl.ds(0, N), :] = h
    h_ref[pl.ds(N, NP - N), :] = jnp.zeros((NP - N, D), jnp.float32)
    as_ref[pl.ds(0, N)] = jnp.sum(h * asrc_ref[...], axis=1)
    as_ref[pl.ds(N, NP - N)] = jnp.zeros((NP - N,), jnp.float32)
    ad_ref[pl.ds(0, N)] = jnp.sum(h * adst_ref[...], axis=1)
    ad_ref[pl.ds(N, NP - N)] = jnp.zeros((NP - N,), jnp.float32)


def _proj(x, w, asrc, adst):
    return pl.pallas_call(
        _proj_body,
        out_shape=[
            jax.ShapeDtypeStruct((NP, D), jnp.float32),
            jax.ShapeDtypeStruct((NP,), jnp.float32),
            jax.ShapeDtypeStruct((NP,), jnp.float32),
        ],
    )(x, w, asrc, adst)


def _norm(o_ref, sp_ref, b_ref):
    s = jnp.sum(sp_ref[...], axis=0)
    acc = o_ref[0] + o_ref[1]
    z = acc * (1.0 / (s + 1e-16))[:, None] + b_ref[...]
    return jnp.maximum(z, 0.0)


def _mid_body(o_ref, sp_ref, b_ref, w_ref, asrc_ref, adst_ref,
              h_ref, as_ref, ad_ref):
    z = _norm(o_ref, sp_ref, b_ref)
    h = jnp.dot(z, w_ref[...], preferred_element_type=jnp.float32)
    h_ref[...] = h
    as_ref[...] = jnp.sum(h * asrc_ref[...], axis=1)
    ad_ref[...] = jnp.sum(h * adst_ref[...], axis=1)


def _mid(o, sp, b, w, asrc, adst):
    return pl.pallas_call(
        _mid_body,
        out_shape=[
            jax.ShapeDtypeStruct((NP, D), jnp.float32),
            jax.ShapeDtypeStruct((NP,), jnp.float32),
            jax.ShapeDtypeStruct((NP,), jnp.float32),
        ],
    )(o, sp, b, w, asrc, adst)


def _final_body(o_ref, sp_ref, b_ref, out_ref):
    out_ref[...] = _norm(o_ref, sp_ref, b_ref)[0:N, :]


def _final(o, sp, b):
    return pl.pallas_call(
        _final_body,
        out_shape=jax.ShapeDtypeStruct((N, D), jnp.float32),
    )(o, sp, b)


_MESH = plsc.VectorSubcoreMesh(core_axis_name="c", subcore_axis_name="s")


def _edge_body(h_hbm, as_hbm, ad_hbm, ei_hbm,
               o_hbm, sp_hbm,
               as_v, ad_v, src_a, src_b, dst_a, dst_b, ex_a, ex_b,
               s_v, rows_a, rows_b, shared,
               sem_sa, sem_sb, sem_da, sem_db, sem_ga, sem_gb,
               sem_ca, sem_cb):
    c = lax.axis_index("c")
    t = lax.axis_index("s")
    w = c * 16 + t
    start = jnp.where(c == 0, t * M0, 16 * M0 + t * M1)
    npairs = jnp.where(c == 0, M0 // 2, M1 // 2)
    m = jnp.where(c == 0, M0, M1)

    pltpu.sync_copy(as_hbm, as_v)
    pltpu.sync_copy(ad_hbm, ad_v)

    zero16 = jnp.zeros((16,), jnp.float32)
    zero16i = jnp.zeros((16,), jnp.int32)
    fullN = jnp.full((16,), N, jnp.int32)

    def _zero_s(i, carry):
        s_v[pl.ds(i * 16, 16)] = zero16
        return carry

    lax.fori_loop(0, NP // 16, _zero_s, 0)

    def _zero_rows(i, carry):
        for cg in range(8):
            rows_a[i, pl.ds(cg * 16, 16)] = zero16
            rows_b[i, pl.ds(cg * 16, 16)] = zero16
        return carry

    lax.fori_loop(0, CHUNK, _zero_rows, 0)

    base = t * ROWS_PER_TILE
    off = 0
    for sz in (64,) * 9 + (ROWS_PER_TILE - 576,):
        pltpu.sync_copy(rows_a.at[pl.ds(0, sz)], shared.at[pl.ds(base + off, sz)])
        off += sz
    plsc.subcore_barrier()

    def _launch_idx(q, row, buf, sem):
        @pl.when(q < NREAL)
        def _():
            pltpu.async_copy(ei_hbm.at[row, pl.ds(q * CHUNK, CHUNK)], buf, sem)

    def _wait_idx(q, row, buf, sem, fill):
        @pl.when(q < NREAL)
        def _():
            pltpu.make_async_copy(
                ei_hbm.at[row, pl.ds(q * CHUNK, CHUNK)], buf, sem).wait()

        @pl.when(q >= NREAL)
        def _():
            for k in range(CHUNK // 16):
                buf[pl.ds(k * 16, 16)] = fill

    def _ex_compute(src_c, dst_c, ex_c):
        for k in range(CHUNK // 16):
            sl = pl.ds(k * 16, 16)
            s16 = src_c[sl]
            d16 = dst_c[sl]
            e = plsc.load_gather(as_v, [s16]) + plsc.load_gather(ad_v, [d16])
            e = jnp.where(e >= 0.0, e, 0.2 * e)
            exv = jnp.exp(e)
            ex_c[sl] = exv
            plsc.addupdate_scatter(s_v, [d16], exv)

    def _multiply(rows, ex_c):
        @plsc.parallel_loop(0, CHUNK, 1, unroll=8)
        def _rowfn(rr):
            exb = plsc.load_gather(ex_c, [jnp.full((16,), rr, jnp.int32)])
            for cg in range(8):
                sl = pl.ds(cg * 16, 16)
                rows[rr, sl] = rows[rr, sl] * exb

    for k in range(CHUNK // 16):
        dst_b[pl.ds(k * 16, 16)] = fullN
    pltpu.async_copy(rows_b, shared.at[dst_b], sem_cb, add=True)
    pltpu.sync_copy(ei_hbm.at[0, pl.ds(start * CHUNK, CHUNK)], src_a)
    pltpu.sync_copy(ei_hbm.at[1, pl.ds(start * CHUNK, CHUNK)], dst_a)
    pltpu.async_copy(h_hbm.at[src_a], rows_a, sem_ga)

    def _pair(i, carry):
        j0 = start + 2 * i
        _launch_idx(j0 + 1, 0, src_b, sem_sb)
        _ex_compute(src_a, dst_a, ex_a)
        pltpu.make_async_copy(rows_b, shared.at[dst_b], sem_cb).wait()
        _launch_idx(j0 + 1, 1, dst_b, sem_db)
        _wait_idx(j0 + 1, 0, src_b, sem_sb, zero16i)
        pltpu.async_copy(h_hbm.at[src_b], rows_b, sem_gb)
        pltpu.make_async_copy(h_hbm.at[src_a], rows_a, sem_ga).wait()
        _multiply(rows_a, ex_a)
        pltpu.async_copy(rows_a, shared.at[dst_a], sem_ca, add=True)

        @pl.when(j0 + 2 < start + m)
        def _():
            _launch_idx(j0 + 2, 0, src_a, sem_sa)

        _wait_idx(j0 + 1, 1, dst_b, sem_db, fullN)
        _ex_compute(src_b, dst_b, ex_b)
        pltpu.make_async_copy(rows_a, shared.at[dst_a], sem_ca).wait()

        @pl.when(j0 + 2 < start + m)
        def _():
            _launch_idx(j0 + 2, 1, dst_a, sem_da)
            _wait_idx(j0 + 2, 0, src_a, sem_sa, zero16i)
            pltpu.async_copy(h_hbm.at[src_a], rows_a, sem_ga)

        pltpu.make_async_copy(h_hbm.at[src_b], rows_b, sem_gb).wait()
        _multiply(rows_b, ex_b)
        pltpu.async_copy(rows_b, shared.at[dst_b], sem_cb, add=True)

        @pl.when(j0 + 2 < start + m)
        def _():
            _wait_idx(j0 + 2, 1, dst_a, sem_da, fullN)

        return carry

    lax.fori_loop(0, npairs, _pair, 0)
    pltpu.make_async_copy(rows_b, shared.at[dst_b], sem_cb).wait()
    plsc.subcore_barrier()

    pltpu.sync_copy(s_v, sp_hbm.at[w])
    off = 0
    for sz in (128, 128, 128, 128, ROWS_PER_TILE - 512):
        pltpu.sync_copy(shared.at[pl.ds(base + off, sz)],
                        o_hbm.at[c, pl.ds(base + off, sz)])
        off += sz


_edge_kernel = pl.kernel(
    _edge_body,
    out_type=[
        jax.ShapeDtypeStruct((2, NP, D), jnp.float32),
        jax.ShapeDtypeStruct((NTILES, NP), jnp.float32),
    ],
    mesh=_MESH,
    compiler_params=pltpu.CompilerParams(needs_layout_passes=False),
    scratch_types=[
        pltpu.VMEM((NP,), jnp.float32),
        pltpu.VMEM((NP,), jnp.float32),
        pltpu.VMEM((CHUNK,), jnp.int32),
        pltpu.VMEM((CHUNK,), jnp.int32),
        pltpu.VMEM((CHUNK,), jnp.int32),
        pltpu.VMEM((CHUNK,), jnp.int32),
        pltpu.VMEM((CHUNK,), jnp.float32),
        pltpu.VMEM((CHUNK,), jnp.float32),
        pltpu.VMEM((NP,), jnp.float32),
        pltpu.VMEM((CHUNK, D), jnp.float32),
        pltpu.VMEM((CHUNK, D), jnp.float32),
        pltpu.VMEM_SHARED((NP, D), jnp.float32),
        pltpu.SemaphoreType.DMA,
        pltpu.SemaphoreType.DMA,
        pltpu.SemaphoreType.DMA,
        pltpu.SemaphoreType.DMA,
        pltpu.SemaphoreType.DMA,
        pltpu.SemaphoreType.DMA,
        pltpu.SemaphoreType.DMA,
        pltpu.SemaphoreType.DMA,
    ],
)


def kernel(x, edge_index, W1, a_src1, a_dst1, b1, W2, a_src2, a_dst2, b2):
    ei = edge_index.astype(jnp.int32)
    h1, as1, ad1 = _proj(x, W1, a_src1.reshape(1, D), a_dst1.reshape(1, D))
    o1, sp1 = _edge_kernel(h1, as1, ad1, ei)
    h2, as2, ad2 = _mid(o1, sp1, b1.reshape(1, D), W2,
                        a_src2.reshape(1, D), a_dst2.reshape(1, D))
    o2, sp2 = _edge_kernel(h2, as2, ad2, ei)
    return _final(o2, sp2, b2.reshape(1, D))
